# Initial kernel scaffold; baseline (speedup 1.0000x reference)
import functools

import jax
import jax.numpy as jnp
from jax import lax
from jax.experimental import pallas as pl
from jax.experimental.pallas import tpu as pltpu
from jax.experimental.pallas import tpu_sc as plsc

B = 16384
HIST = 50
F = 64
EPS_BN = 1e-5
EPS_LN = 1e-5

NC = 2
NS = 16
L = 16
NW = NC * NS
BPW = B // NW
CH = 8
NCHUNK = BPW // CH
GROWS = CH * HIST


def _sc_mesh():
    return plsc.VectorSubcoreMesh(
        core_axis_name="c", subcore_axis_name="s", num_cores=NC, num_subcores=NS
    )


@functools.partial(
    pl.kernel,
    out_type=jax.ShapeDtypeStruct((B, F), jnp.float32),
    mesh=_sc_mesh(),
    scratch_types=[
        pltpu.VMEM((BPW * HIST,), jnp.int32),
        pltpu.VMEM((2, GROWS, F), jnp.float32),
        pltpu.VMEM((BPW, F), jnp.float32),
        pltpu.SemaphoreType.DMA,
        pltpu.SemaphoreType.DMA,
    ],
)
def _gather_pool(idx_hbm, table_hbm, out_hbm, idx_v, rows_v, acc_v, sem0, sem1):
    wid = lax.axis_index("s") * NC + lax.axis_index("c")
    base = wid * BPW

    pltpu.sync_copy(idx_hbm.at[wid], idx_v)

    sems = (sem0, sem1)

    def gather(c, slot):
        return pltpu.make_async_copy(
            table_hbm.at[idx_v.at[pl.ds(c * GROWS, GROWS)]],
            rows_v.at[slot],
            sems[slot],
        )

    def accumulate(c, slot):
        def row_body(r, _):
            for cc in range(F // L):
                sl = pl.ds(cc * L, L)
                v = rows_v[slot, r * HIST, sl]
                for j in range(1, HIST):
                    v = v + rows_v[slot, r * HIST + j, sl]
            acc_v[c * CH + r, sl] = v
            return 0

        lax.fori_loop(0, CH, row_body, 0)

    gather(0, 0).start()

    def chunk_body(c2, _):
        c = c2 * 2
        gather(c + 1, 1).start()
        gather(c, 0).wait()
        accumulate(c, 0)

        @pl.when(c + 2 < NCHUNK)
        def _():
            gather(c + 2, 0).start()

        gather(c + 1, 1).wait()
        accumulate(c + 1, 1)
        return 0

    lax.fori_loop(0, NCHUNK // 2, chunk_body, 0)

    pltpu.sync_copy(acc_v, out_hbm.at[pl.ds(base, BPW)])


def _dense_body(x_ref, w_ref, b_ref, bng_ref, bnb_ref, lng_ref, lnb_ref, o_ref):
    x = x_ref[...] * (1.0 / HIST)
    h = lax.dot_general(
        x, w_ref[...], (((1,), (1,)), ((), ())),
        preferred_element_type=jnp.float32,
    )
    h = h + b_ref[...]
    mu = jnp.mean(h, axis=0, keepdims=True)
    d = h - mu
    var = jnp.mean(d * d, axis=0, keepdims=True)
    h = d * lax.rsqrt(var + EPS_BN) * bng_ref[...] + bnb_ref[...]
    lmu = jnp.mean(h, axis=1, keepdims=True)
    ld = h - lmu
    lvar = jnp.mean(ld * ld, axis=1, keepdims=True)
    o_ref[...] = ld * lax.rsqrt(lvar + EPS_LN) * lng_ref[...] + lnb_ref[...]


def kernel(inputs, table, W, b, bn_gamma, bn_beta, ln_gamma, ln_beta):
    idx = inputs.astype(jnp.int32).reshape(NW, BPW * HIST)
    pooled_sum = _gather_pool(idx, table)
    out = pl.pallas_call(
        _dense_body,
        out_shape=jax.ShapeDtypeStruct((B, F), jnp.float32),
    )(
        pooled_sum,
        W,
        b.reshape(1, F),
        bn_gamma.reshape(1, F),
        bn_beta.reshape(1, F),
        ln_gamma.reshape(1, F),
        ln_beta.reshape(1, F),
    )
    return out


# speedup vs baseline: 2.5206x; 2.5206x over previous
import functools

import jax
import jax.numpy as jnp
from jax import lax
from jax.experimental import pallas as pl
from jax.experimental.pallas import tpu as pltpu
from jax.experimental.pallas import tpu_sc as plsc

B = 16384
HIST = 50
F = 64
EPS_BN = 1e-5
EPS_LN = 1e-5

NC = 2
NS = 16
L = 16
NW = NC * NS
BPW = B // NW
CH = 8
NCHUNK = BPW // CH
GROWS = CH * HIST


def _sc_mesh():
    return plsc.VectorSubcoreMesh(
        core_axis_name="c", subcore_axis_name="s", num_cores=NC, num_subcores=NS
    )


@functools.partial(
    pl.kernel,
    out_type=jax.ShapeDtypeStruct((B, F), jnp.float32),
    mesh=_sc_mesh(),
    scratch_types=[
        pltpu.VMEM((BPW * HIST,), jnp.int32),
        pltpu.VMEM((2, GROWS, F), jnp.float32),
        pltpu.VMEM((BPW, F), jnp.float32),
        pltpu.SemaphoreType.DMA,
        pltpu.SemaphoreType.DMA,
    ],
    compiler_params=pltpu.CompilerParams(use_tc_tiling_on_sc=False),
)
def _gather_pool(idx_hbm, table_hbm, out_hbm, idx_v, rows_v, acc_v, sem0, sem1):
    wid = lax.axis_index("s") * NC + lax.axis_index("c")
    base = wid * BPW

    pltpu.sync_copy(idx_hbm.at[wid], idx_v)

    sems = (sem0, sem1)

    def gather(c, slot):
        return pltpu.make_async_copy(
            table_hbm.at[idx_v.at[pl.ds(c * GROWS, GROWS)]],
            rows_v.at[slot],
            sems[slot],
        )

    def accumulate(c, slot):
        def row_body(r, _):
            for cc in range(F // L):
                sl = pl.ds(cc * L, L)
                v = rows_v[slot, r * HIST, sl]
                for j in range(1, HIST):
                    v = v + rows_v[slot, r * HIST + j, sl]
                acc_v[c * CH + r, sl] = v
            return 0

        lax.fori_loop(0, CH, row_body, 0)

    gather(0, 0).start()

    def chunk_body(c2, _):
        c = c2 * 2
        gather(c + 1, 1).start()
        gather(c, 0).wait()
        accumulate(c, 0)

        @pl.when(c + 2 < NCHUNK)
        def _():
            gather(c + 2, 0).start()

        gather(c + 1, 1).wait()
        accumulate(c + 1, 1)
        return 0

    lax.fori_loop(0, NCHUNK // 2, chunk_body, 0)

    pltpu.sync_copy(acc_v, out_hbm.at[pl.ds(base, BPW)])


def _dense_body(x_ref, w_ref, b_ref, bng_ref, bnb_ref, lng_ref, lnb_ref, o_ref):
    x = x_ref[...] * (1.0 / HIST)
    h = lax.dot_general(
        x, w_ref[...], (((1,), (1,)), ((), ())),
        preferred_element_type=jnp.float32,
    )
    h = h + b_ref[...]
    mu = jnp.mean(h, axis=0, keepdims=True)
    d = h - mu
    var = jnp.mean(d * d, axis=0, keepdims=True)
    h = d * lax.rsqrt(var + EPS_BN) * bng_ref[...] + bnb_ref[...]
    lmu = jnp.mean(h, axis=1, keepdims=True)
    ld = h - lmu
    lvar = jnp.mean(ld * ld, axis=1, keepdims=True)
    o_ref[...] = ld * lax.rsqrt(lvar + EPS_LN) * lng_ref[...] + lnb_ref[...]


def kernel(inputs, table, W, b, bn_gamma, bn_beta, ln_gamma, ln_beta):
    idx = inputs.astype(jnp.int32).reshape(NW, BPW * HIST)
    pooled_sum = _gather_pool(idx, table)
    out = pl.pallas_call(
        _dense_body,
        out_shape=jax.ShapeDtypeStruct((B, F), jnp.float32),
    )(
        pooled_sum,
        W,
        b.reshape(1, F),
        bn_gamma.reshape(1, F),
        bn_beta.reshape(1, F),
        ln_gamma.reshape(1, F),
        ln_beta.reshape(1, F),
    )
    return out
